# baseline (speedup 1.0000x reference)
import functools

import jax
import jax.numpy as jnp
from jax import lax
from jax.experimental import pallas as pl
from jax.experimental.pallas import tpu as pltpu
from jax.experimental.pallas import tpu_sc as plsc

N_NODES = 10000
D_HID = 128
NC = 2
NS = 16
NW = NC * NS
CHUNK = 64
N_ACC = 10112
ROWS_PER_TILE = N_ACC // NS
DEG_PAD = 16384
DEG_PER_TILE = DEG_PAD // NS
BM = 1000
NBUF = 4

_MESH = plsc.VectorSubcoreMesh(
    core_axis_name="c", subcore_axis_name="s", num_cores=NC, num_subcores=NS)


def _sc_degree(dst_r, ew_r, steps):
  @functools.partial(
      pl.kernel,
      out_type=jax.ShapeDtypeStruct((NC * DEG_PAD,), jnp.float32),
      mesh=_MESH,
      scratch_types=[
          pltpu.VMEM_SHARED((DEG_PAD,), jnp.float32),
          pltpu.VMEM((steps, CHUNK), jnp.int32),
          pltpu.VMEM((steps, CHUNK), jnp.float32),
          pltpu.VMEM((DEG_PER_TILE,), jnp.float32),
      ] + [pltpu.SemaphoreType.DMA for _ in range(8)],
  )
  def k(dst_hbm, ew_hbm, out_hbm, deg_sh, dst_all, ew_all, zbuf, *asem):
    c = lax.axis_index("c")
    s = lax.axis_index("s")
    wid = s * NC + c
    zero = jnp.zeros((16,), jnp.float32)

    pltpu.sync_copy(dst_hbm.at[wid], dst_all)
    pltpu.sync_copy(ew_hbm.at[wid], ew_all)

    @pl.loop(0, DEG_PER_TILE // 16)
    def _(i):
      zbuf[pl.ds(i * 16, 16)] = zero

    base = s * DEG_PER_TILE
    pltpu.sync_copy(zbuf, deg_sh.at[pl.ds(base, DEG_PER_TILE)])
    plsc.subcore_barrier()

    @pl.loop(0, steps, step=8)
    def _(t0):
      for b in range(8):
        @pl.when(t0 >= 8)
        def _():
          pltpu.make_async_copy(ew_all.at[t0 + b - 8],
                                deg_sh.at[dst_all.at[t0 + b - 8]],
                                asem[b]).wait()
        pltpu.async_copy(ew_all.at[t0 + b], deg_sh.at[dst_all.at[t0 + b]],
                         asem[b], add=True)

    for b in range(8):
      pltpu.make_async_copy(ew_all.at[steps - 8 + b],
                            deg_sh.at[dst_all.at[steps - 8 + b]],
                            asem[b]).wait()

    plsc.subcore_barrier()
    obase = pl.multiple_of(c * DEG_PAD + base, DEG_PER_TILE)
    pltpu.sync_copy(deg_sh.at[pl.ds(base, DEG_PER_TILE)],
                    out_hbm.at[pl.ds(obase, DEG_PER_TILE)])

  return k(dst_r, ew_r)


T0_CHUNKS = 256
T1_CHUNKS = 64


def _sc_aggregate(g, src_p, dst_r, ew_r, steps):
  tmax = max(T0_CHUNKS, T1_CHUNKS)
  zchunks = [(i * 64, 64) for i in range(9)] + [(576, 56)]

  @functools.partial(
      pl.kernel,
      out_type=jax.ShapeDtypeStruct((NC, N_ACC, D_HID), jnp.float32),
      mesh=_MESH,
      scratch_types=[
          pltpu.VMEM_SHARED((N_ACC, D_HID), jnp.float32),
          pltpu.VMEM((tmax * CHUNK,), jnp.int32),
      ] + [pltpu.VMEM((CHUNK,), jnp.int32) for _ in range(NBUF)]
        + [pltpu.VMEM((CHUNK,), jnp.float32) for _ in range(NBUF)]
        + [pltpu.VMEM((CHUNK, D_HID), jnp.float32) for _ in range(NBUF)]
        + [pltpu.SemaphoreType.DMA for _ in range(4 * NBUF)],
  )
  def k(g_hbm, src_hbm, dst_hbm, ew_hbm, out_hbm, acc_sh, src_all,
        *bufs_and_sems):
    dstb = bufs_and_sems[:NBUF]
    ewb = bufs_and_sems[NBUF:2 * NBUF]
    rows = bufs_and_sems[2 * NBUF:3 * NBUF]
    gsem = bufs_and_sems[3 * NBUF:4 * NBUF]
    ssem = bufs_and_sems[4 * NBUF:5 * NBUF]
    dsem = bufs_and_sems[5 * NBUF:6 * NBUF]
    esem = bufs_and_sems[6 * NBUF:]
    c = lax.axis_index("c")
    s = lax.axis_index("s")
    zero = jnp.zeros((16,), jnp.float32)

    @pl.loop(0, CHUNK)
    def _(i):
      for j in range(D_HID // 16):
        rows[0][i, pl.ds(j * 16, 16)] = zero

    rbase = s * ROWS_PER_TILE
    for o, sz in zchunks:
      pltpu.sync_copy(rows[0].at[pl.ds(0, sz)],
                      acc_sh.at[pl.ds(pl.multiple_of(rbase + o, 8), sz)])

    def pipeline(chbase, nsteps):
      pltpu.sync_copy(
          src_hbm.at[pl.ds(pl.multiple_of(chbase * CHUNK, CHUNK),
                           nsteps * CHUNK)],
          src_all.at[pl.ds(0, nsteps * CHUNK)])

      for b in range(NBUF - 1):
        pltpu.async_copy(dst_hbm.at[chbase + b], dstb[b], dsem[b])
        pltpu.async_copy(ew_hbm.at[chbase + b], ewb[b], esem[b])
        pltpu.async_copy(
            g_hbm.at[src_all.at[pl.ds(b * CHUNK, CHUNK)]], rows[b], gsem[b])

      plsc.subcore_barrier()

      @pl.loop(0, nsteps, step=NBUF)
      def _(t0):
        for b in range(NBUF):
          t = t0 + b
          ch = chbase + t
          pltpu.make_async_copy(
              g_hbm.at[src_all.at[pl.ds(t * CHUNK, CHUNK)]], rows[b],
              gsem[b]).wait()
          pltpu.make_async_copy(ew_hbm.at[ch], ewb[b], esem[b]).wait()

          @pl.loop(0, CHUNK // 16)
          def _(gg):
            wv = ewb[b][pl.ds(gg * 16, 16)]
            for kk in range(16):
              w = wv[kk]
              e = gg * 16 + kk
              for j in range(D_HID // 16):
                sl = pl.ds(j * 16, 16)
                rows[b][e, sl] = rows[b][e, sl] * w

          pltpu.make_async_copy(dst_hbm.at[ch], dstb[b], dsem[b]).wait()
          pltpu.async_copy(rows[b], acc_sh.at[dstb[b]], ssem[b], add=True)

          bf = (b + NBUF - 1) % NBUF

          @pl.when(t + NBUF - 1 < nsteps)
          def _():
            @pl.when(t >= 1)
            def _():
              pltpu.make_async_copy(rows[bf], acc_sh.at[dstb[bf]],
                                    ssem[bf]).wait()
            pltpu.async_copy(dst_hbm.at[ch + NBUF - 1], dstb[bf], dsem[bf])
            pltpu.async_copy(ew_hbm.at[ch + NBUF - 1], ewb[bf], esem[bf])
            pltpu.async_copy(
                g_hbm.at[src_all.at[pl.ds((t + NBUF - 1) * CHUNK, CHUNK)]],
                rows[bf], gsem[bf])

      for b in range(NBUF):
        pltpu.make_async_copy(rows[b], acc_sh.at[dstb[b]], ssem[b]).wait()

    @pl.when(c == 0)
    def _():
      pipeline(s * T0_CHUNKS, T0_CHUNKS)

    @pl.when(c == 1)
    def _():
      pipeline(NS * T0_CHUNKS + s * T1_CHUNKS, T1_CHUNKS)

    plsc.subcore_barrier()
    for o, sz in zchunks:
      off2 = pl.multiple_of(rbase + o, 8)
      pltpu.sync_copy(acc_sh.at[pl.ds(off2, sz)],
                      out_hbm.at[c, pl.ds(off2, sz)])

  return k(g, src_p, dst_r, ew_r)


def _tc_prologue(x, W1, degp):
  n, d_in = x.shape

  def body(x_ref, w_ref, deg_ref, g_ref, dinv_ref):
    deg = deg_ref[0] + deg_ref[1] + 1.0
    dinv = jnp.where(deg > 0, lax.rsqrt(deg), 0.0)
    h = jnp.dot(x_ref[...], w_ref[...], preferred_element_type=jnp.float32)
    g_ref[...] = h * dinv
    dinv_ref[...] = dinv

  return pl.pallas_call(
      body,
      grid=(n // BM,),
      in_specs=[
          pl.BlockSpec((BM, d_in), lambda i: (i, 0)),
          pl.BlockSpec((d_in, D_HID), lambda i: (0, 0)),
          pl.BlockSpec((NC, BM, 1), lambda i: (0, i, 0)),
      ],
      out_specs=[
          pl.BlockSpec((BM, D_HID), lambda i: (i, 0)),
          pl.BlockSpec((BM, 1), lambda i: (i, 0)),
      ],
      out_shape=[
          jax.ShapeDtypeStruct((n, D_HID), jnp.float32),
          jax.ShapeDtypeStruct((n, 1), jnp.float32),
      ],
  )(x, W1, degp)


def _tc_mid(accp, g, dinv, b, W_next):
  n = g.shape[0]

  def body(acc_ref, g_ref, dinv_ref, b_ref, w_ref, out_ref):
    dinv = dinv_ref[...]
    t = dinv * (acc_ref[0] + acc_ref[1] + g_ref[...]) + b_ref[...]
    z = jnp.maximum(t, 0.0)
    h = jnp.dot(z, w_ref[...], preferred_element_type=jnp.float32)
    out_ref[...] = h * dinv

  return pl.pallas_call(
      body,
      grid=(n // BM,),
      in_specs=[
          pl.BlockSpec((NC, BM, D_HID), lambda i: (0, i, 0)),
          pl.BlockSpec((BM, D_HID), lambda i: (i, 0)),
          pl.BlockSpec((BM, 1), lambda i: (i, 0)),
          pl.BlockSpec((1, D_HID), lambda i: (0, 0)),
          pl.BlockSpec((D_HID, D_HID), lambda i: (0, 0)),
      ],
      out_specs=pl.BlockSpec((BM, D_HID), lambda i: (i, 0)),
      out_shape=jax.ShapeDtypeStruct((n, D_HID), jnp.float32),
  )(accp, g, dinv, b, W_next)


def _tc_final(accp, g, dinv, b):
  n = g.shape[0]

  def body(acc_ref, g_ref, dinv_ref, b_ref, out_ref):
    t = dinv_ref[...] * (acc_ref[0] + acc_ref[1] + g_ref[...]) + b_ref[...]
    m = jnp.max(t, axis=1, keepdims=True)
    sh = t - m
    lse = jnp.log(jnp.sum(jnp.exp(sh), axis=1, keepdims=True))
    out_ref[...] = sh - lse

  return pl.pallas_call(
      body,
      grid=(n // BM,),
      in_specs=[
          pl.BlockSpec((NC, BM, D_HID), lambda i: (0, i, 0)),
          pl.BlockSpec((BM, D_HID), lambda i: (i, 0)),
          pl.BlockSpec((BM, 1), lambda i: (i, 0)),
          pl.BlockSpec((1, D_HID), lambda i: (0, 0)),
      ],
      out_specs=pl.BlockSpec((BM, D_HID), lambda i: (i, 0)),
      out_shape=jax.ShapeDtypeStruct((n, D_HID), jnp.float32),
  )(accp, g, dinv, b)


def kernel(x, edge_index, edge_attr, W1, b1, W2, b2, W3, b3, W4, b4):
  e = edge_index.shape[1]
  steps = -(-e // (NW * CHUNK))
  steps = -(-steps // NBUF) * NBUF
  ep = steps * NW * CHUNK
  pad = ep - e
  src_p = jnp.concatenate([edge_index[0], jnp.zeros((pad,), jnp.int32)])
  dst_p = jnp.concatenate([edge_index[1], jnp.zeros((pad,), jnp.int32)])
  ew_p = jnp.concatenate([edge_attr, jnp.zeros((pad,), jnp.float32)])
  dst_r = dst_p.reshape(NW * steps, CHUNK)
  ew_r = ew_p.reshape(NW * steps, CHUNK)

  degp = _sc_degree(dst_r.reshape(NW, steps, CHUNK), ew_r.reshape(NW, steps, CHUNK), steps)
  degp = degp.reshape(NC, DEG_PAD)[:, :N_NODES, None]

  g, dinv = _tc_prologue(x, W1, degp)
  for b, w_next in ((b1, W2), (b2, W3), (b3, W4)):
    acc = _sc_aggregate(g, src_p, dst_r, ew_r, steps)
    g = _tc_mid(acc, g, dinv, b.reshape(1, D_HID), w_next)
  acc = _sc_aggregate(g, src_p, dst_r, ew_r, steps)
  return _tc_final(acc, g, dinv, b4.reshape(1, D_HID))


# speedup vs baseline: 1.0025x; 1.0025x over previous
import functools

import jax
import jax.numpy as jnp
from jax import lax
from jax.experimental import pallas as pl
from jax.experimental.pallas import tpu as pltpu
from jax.experimental.pallas import tpu_sc as plsc

N_NODES = 10000
D_HID = 128
NC = 2
NS = 16
NW = NC * NS
CHUNK = 64
N_ACC = 10112
ROWS_PER_TILE = N_ACC // NS
DEG_PAD = 16384
DEG_PER_TILE = DEG_PAD // NS
BM = 1000
NBUF = 4

_MESH = plsc.VectorSubcoreMesh(
    core_axis_name="c", subcore_axis_name="s", num_cores=NC, num_subcores=NS)


def _sc_degree(dst_r, ew_r, steps):
  @functools.partial(
      pl.kernel,
      out_type=jax.ShapeDtypeStruct((NC * DEG_PAD,), jnp.float32),
      mesh=_MESH,
      scratch_types=[
          pltpu.VMEM_SHARED((DEG_PAD,), jnp.float32),
          pltpu.VMEM((steps, CHUNK), jnp.int32),
          pltpu.VMEM((steps, CHUNK), jnp.float32),
          pltpu.VMEM((DEG_PER_TILE,), jnp.float32),
      ] + [pltpu.SemaphoreType.DMA for _ in range(8)],
  )
  def k(dst_hbm, ew_hbm, out_hbm, deg_sh, dst_all, ew_all, zbuf, *asem):
    c = lax.axis_index("c")
    s = lax.axis_index("s")
    wid = s * NC + c
    zero = jnp.zeros((16,), jnp.float32)

    pltpu.sync_copy(dst_hbm.at[wid], dst_all)
    pltpu.sync_copy(ew_hbm.at[wid], ew_all)

    @pl.loop(0, DEG_PER_TILE // 16)
    def _(i):
      zbuf[pl.ds(i * 16, 16)] = zero

    base = s * DEG_PER_TILE
    pltpu.sync_copy(zbuf, deg_sh.at[pl.ds(base, DEG_PER_TILE)])
    plsc.subcore_barrier()

    @pl.loop(0, steps, step=8)
    def _(t0):
      for b in range(8):
        @pl.when(t0 >= 8)
        def _():
          pltpu.make_async_copy(ew_all.at[t0 + b - 8],
                                deg_sh.at[dst_all.at[t0 + b - 8]],
                                asem[b]).wait()
        pltpu.async_copy(ew_all.at[t0 + b], deg_sh.at[dst_all.at[t0 + b]],
                         asem[b], add=True)

    for b in range(8):
      pltpu.make_async_copy(ew_all.at[steps - 8 + b],
                            deg_sh.at[dst_all.at[steps - 8 + b]],
                            asem[b]).wait()

    plsc.subcore_barrier()
    obase = pl.multiple_of(c * DEG_PAD + base, DEG_PER_TILE)
    pltpu.sync_copy(deg_sh.at[pl.ds(base, DEG_PER_TILE)],
                    out_hbm.at[pl.ds(obase, DEG_PER_TILE)])

  return k(dst_r, ew_r)


T0_CHUNKS = 240
T1_CHUNKS = 80


def _sc_aggregate(g, src_p, dst_r, ew_r, steps):
  tmax = max(T0_CHUNKS, T1_CHUNKS)
  zchunks = [(i * 64, 64) for i in range(9)] + [(576, 56)]

  @functools.partial(
      pl.kernel,
      out_type=jax.ShapeDtypeStruct((NC, N_ACC, D_HID), jnp.float32),
      mesh=_MESH,
      scratch_types=[
          pltpu.VMEM_SHARED((N_ACC, D_HID), jnp.float32),
          pltpu.VMEM((tmax * CHUNK,), jnp.int32),
      ] + [pltpu.VMEM((CHUNK,), jnp.int32) for _ in range(NBUF)]
        + [pltpu.VMEM((CHUNK,), jnp.float32) for _ in range(NBUF)]
        + [pltpu.VMEM((CHUNK, D_HID), jnp.float32) for _ in range(NBUF)]
        + [pltpu.SemaphoreType.DMA for _ in range(4 * NBUF)],
  )
  def k(g_hbm, src_hbm, dst_hbm, ew_hbm, out_hbm, acc_sh, src_all,
        *bufs_and_sems):
    dstb = bufs_and_sems[:NBUF]
    ewb = bufs_and_sems[NBUF:2 * NBUF]
    rows = bufs_and_sems[2 * NBUF:3 * NBUF]
    gsem = bufs_and_sems[3 * NBUF:4 * NBUF]
    ssem = bufs_and_sems[4 * NBUF:5 * NBUF]
    dsem = bufs_and_sems[5 * NBUF:6 * NBUF]
    esem = bufs_and_sems[6 * NBUF:]
    c = lax.axis_index("c")
    s = lax.axis_index("s")
    zero = jnp.zeros((16,), jnp.float32)

    @pl.loop(0, CHUNK)
    def _(i):
      for j in range(D_HID // 16):
        rows[0][i, pl.ds(j * 16, 16)] = zero

    rbase = s * ROWS_PER_TILE
    for o, sz in zchunks:
      pltpu.sync_copy(rows[0].at[pl.ds(0, sz)],
                      acc_sh.at[pl.ds(pl.multiple_of(rbase + o, 8), sz)])

    def pipeline(chbase, nsteps):
      pltpu.sync_copy(
          src_hbm.at[pl.ds(pl.multiple_of(chbase * CHUNK, CHUNK),
                           nsteps * CHUNK)],
          src_all.at[pl.ds(0, nsteps * CHUNK)])

      for b in range(NBUF - 1):
        pltpu.async_copy(dst_hbm.at[chbase + b], dstb[b], dsem[b])
        pltpu.async_copy(ew_hbm.at[chbase + b], ewb[b], esem[b])
        pltpu.async_copy(
            g_hbm.at[src_all.at[pl.ds(b * CHUNK, CHUNK)]], rows[b], gsem[b])

      plsc.subcore_barrier()

      @pl.loop(0, nsteps, step=NBUF)
      def _(t0):
        for b in range(NBUF):
          t = t0 + b
          ch = chbase + t
          pltpu.make_async_copy(
              g_hbm.at[src_all.at[pl.ds(t * CHUNK, CHUNK)]], rows[b],
              gsem[b]).wait()
          pltpu.make_async_copy(ew_hbm.at[ch], ewb[b], esem[b]).wait()

          @pl.loop(0, CHUNK // 16)
          def _(gg):
            wv = ewb[b][pl.ds(gg * 16, 16)]
            for kk in range(16):
              w = wv[kk]
              e = gg * 16 + kk
              for j in range(D_HID // 16):
                sl = pl.ds(j * 16, 16)
                rows[b][e, sl] = rows[b][e, sl] * w

          pltpu.make_async_copy(dst_hbm.at[ch], dstb[b], dsem[b]).wait()
          pltpu.async_copy(rows[b], acc_sh.at[dstb[b]], ssem[b], add=True)

          bf = (b + NBUF - 1) % NBUF

          @pl.when(t + NBUF - 1 < nsteps)
          def _():
            @pl.when(t >= 1)
            def _():
              pltpu.make_async_copy(rows[bf], acc_sh.at[dstb[bf]],
                                    ssem[bf]).wait()
            pltpu.async_copy(dst_hbm.at[ch + NBUF - 1], dstb[bf], dsem[bf])
            pltpu.async_copy(ew_hbm.at[ch + NBUF - 1], ewb[bf], esem[bf])
            pltpu.async_copy(
                g_hbm.at[src_all.at[pl.ds((t + NBUF - 1) * CHUNK, CHUNK)]],
                rows[bf], gsem[bf])

      for b in range(NBUF):
        pltpu.make_async_copy(rows[b], acc_sh.at[dstb[b]], ssem[b]).wait()

    @pl.when(c == 0)
    def _():
      pipeline(s * T0_CHUNKS, T0_CHUNKS)

    @pl.when(c == 1)
    def _():
      pipeline(NS * T0_CHUNKS + s * T1_CHUNKS, T1_CHUNKS)

    plsc.subcore_barrier()
    for o, sz in zchunks:
      off2 = pl.multiple_of(rbase + o, 8)
      pltpu.sync_copy(acc_sh.at[pl.ds(off2, sz)],
                      out_hbm.at[c, pl.ds(off2, sz)])

  return k(g, src_p, dst_r, ew_r)


def _tc_prologue(x, W1, degp):
  n, d_in = x.shape

  def body(x_ref, w_ref, deg_ref, g_ref, dinv_ref):
    deg = deg_ref[0] + deg_ref[1] + 1.0
    dinv = jnp.where(deg > 0, lax.rsqrt(deg), 0.0)
    h = jnp.dot(x_ref[...], w_ref[...], preferred_element_type=jnp.float32)
    g_ref[...] = h * dinv
    dinv_ref[...] = dinv

  return pl.pallas_call(
      body,
      grid=(n // BM,),
      in_specs=[
          pl.BlockSpec((BM, d_in), lambda i: (i, 0)),
          pl.BlockSpec((d_in, D_HID), lambda i: (0, 0)),
          pl.BlockSpec((NC, BM, 1), lambda i: (0, i, 0)),
      ],
      out_specs=[
          pl.BlockSpec((BM, D_HID), lambda i: (i, 0)),
          pl.BlockSpec((BM, 1), lambda i: (i, 0)),
      ],
      out_shape=[
          jax.ShapeDtypeStruct((n, D_HID), jnp.float32),
          jax.ShapeDtypeStruct((n, 1), jnp.float32),
      ],
  )(x, W1, degp)


def _tc_mid(accp, g, dinv, b, W_next):
  n = g.shape[0]

  def body(acc_ref, g_ref, dinv_ref, b_ref, w_ref, out_ref):
    dinv = dinv_ref[...]
    t = dinv * (acc_ref[0] + acc_ref[1] + g_ref[...]) + b_ref[...]
    z = jnp.maximum(t, 0.0)
    h = jnp.dot(z, w_ref[...], preferred_element_type=jnp.float32)
    out_ref[...] = h * dinv

  return pl.pallas_call(
      body,
      grid=(n // BM,),
      in_specs=[
          pl.BlockSpec((NC, BM, D_HID), lambda i: (0, i, 0)),
          pl.BlockSpec((BM, D_HID), lambda i: (i, 0)),
          pl.BlockSpec((BM, 1), lambda i: (i, 0)),
          pl.BlockSpec((1, D_HID), lambda i: (0, 0)),
          pl.BlockSpec((D_HID, D_HID), lambda i: (0, 0)),
      ],
      out_specs=pl.BlockSpec((BM, D_HID), lambda i: (i, 0)),
      out_shape=jax.ShapeDtypeStruct((n, D_HID), jnp.float32),
  )(accp, g, dinv, b, W_next)


def _tc_final(accp, g, dinv, b):
  n = g.shape[0]

  def body(acc_ref, g_ref, dinv_ref, b_ref, out_ref):
    t = dinv_ref[...] * (acc_ref[0] + acc_ref[1] + g_ref[...]) + b_ref[...]
    m = jnp.max(t, axis=1, keepdims=True)
    sh = t - m
    lse = jnp.log(jnp.sum(jnp.exp(sh), axis=1, keepdims=True))
    out_ref[...] = sh - lse

  return pl.pallas_call(
      body,
      grid=(n // BM,),
      in_specs=[
          pl.BlockSpec((NC, BM, D_HID), lambda i: (0, i, 0)),
          pl.BlockSpec((BM, D_HID), lambda i: (i, 0)),
          pl.BlockSpec((BM, 1), lambda i: (i, 0)),
          pl.BlockSpec((1, D_HID), lambda i: (0, 0)),
      ],
      out_specs=pl.BlockSpec((BM, D_HID), lambda i: (i, 0)),
      out_shape=jax.ShapeDtypeStruct((n, D_HID), jnp.float32),
  )(accp, g, dinv, b)


def kernel(x, edge_index, edge_attr, W1, b1, W2, b2, W3, b3, W4, b4):
  e = edge_index.shape[1]
  steps = -(-e // (NW * CHUNK))
  steps = -(-steps // NBUF) * NBUF
  ep = steps * NW * CHUNK
  pad = ep - e
  src_p = jnp.concatenate([edge_index[0], jnp.zeros((pad,), jnp.int32)])
  dst_p = jnp.concatenate([edge_index[1], jnp.zeros((pad,), jnp.int32)])
  ew_p = jnp.concatenate([edge_attr, jnp.zeros((pad,), jnp.float32)])
  dst_r = dst_p.reshape(NW * steps, CHUNK)
  ew_r = ew_p.reshape(NW * steps, CHUNK)

  degp = _sc_degree(dst_r.reshape(NW, steps, CHUNK), ew_r.reshape(NW, steps, CHUNK), steps)
  degp = degp.reshape(NC, DEG_PAD)[:, :N_NODES, None]

  g, dinv = _tc_prologue(x, W1, degp)
  for b, w_next in ((b1, W2), (b2, W3), (b3, W4)):
    acc = _sc_aggregate(g, src_p, dst_r, ew_r, steps)
    g = _tc_mid(acc, g, dinv, b.reshape(1, D_HID), w_next)
  acc = _sc_aggregate(g, src_p, dst_r, ew_r, steps)
  return _tc_final(acc, g, dinv, b4.reshape(1, D_HID))
